# Initial kernel scaffold; baseline (speedup 1.0000x reference)
import jax
import jax.numpy as jnp
from jax.experimental import pallas as pl


def kernel(data, batch, W1a, b1a, W1b, b1b, W1c, b1c, W2, b2, Wl, bl, Wm1, bm1, Wm2, bm2, Wm3, bm3):
    raise NotImplementedError("write your pallas kernel here")


# speedup vs baseline: 14.2418x; 14.2418x over previous
import functools

import jax
import jax.numpy as jnp
from jax.experimental import pallas as pl

B = 8
P = 1024
K = 20

_f32 = jnp.float32


def _dot(a, b):
    return jax.lax.dot_general(a, b, (((1,), (0,)), ((), ())),
                               preferred_element_type=_f32)


def _topk_idx(d2, k):
    iota0 = jax.lax.broadcasted_iota(jnp.int32, (P, P), 0)
    rows = []
    for _ in range(k):
        m = jnp.min(d2, axis=0, keepdims=True)
        am = jnp.min(jnp.where(d2 == m, iota0, P * 2), axis=0,
                     keepdims=True)
        rows.append(am)
        d2 = jnp.where(iota0 == am, _f32(1e30), d2)
    return jnp.concatenate(rows, axis=0)


def _pairwise_d2(xt):
    xx = jax.lax.dot_general(xt, xt, (((0,), (0,)), ((), ())),
                             preferred_element_type=_f32)
    sq = xt * xt
    ones_c = jnp.ones((xt.shape[0], 1), _f32)
    ones_r = jnp.ones((1, xt.shape[0]), _f32)
    sqc = jax.lax.dot_general(sq, ones_c, (((0,), (0,)), ((), ())),
                              preferred_element_type=_f32)
    sqr = _dot(ones_r, sq)
    d2 = sqc + sqr - 2.0 * xx
    iota0 = jax.lax.broadcasted_iota(jnp.int32, (P, P), 0)
    iota1 = jax.lax.broadcasted_iota(jnp.int32, (P, P), 1)
    return d2 + jnp.where(iota0 == iota1, _f32(1e10), _f32(0.0))


def _cloud_kernel(xt_ref, wu1_ref, wv1_ref, b1a_ref, w1b_ref, b1b_ref,
                  w1c_ref, b1c_ref, w2u_ref, w2v_ref, b2_ref,
                  wl_ref, bl_ref, pool_ref):
    xt = xt_ref[0]
    iota0 = jax.lax.broadcasted_iota(jnp.int32, (P, P), 0)

    idx1 = _topk_idx(_pairwise_d2(xt), K)
    u1 = _dot(wu1_ref[...], xt) + b1a_ref[...]
    acc1 = None
    for k in range(K):
        oh = (iota0 == idx1[k:k + 1, :]).astype(_f32)
        xj = _dot(xt, oh)
        h = jnp.maximum(u1 + _dot(wv1_ref[...], xj), 0.0)
        h = jnp.maximum(_dot(w1b_ref[...], h) + b1b_ref[...], 0.0)
        h = _dot(w1c_ref[...], h) + b1c_ref[...]
        acc1 = h if acc1 is None else jnp.maximum(acc1, h)
    x1t = acc1

    idx2 = _topk_idx(_pairwise_d2(x1t), K)
    u2 = _dot(w2u_ref[...], x1t) + b2_ref[...]
    acc2 = None
    for k in range(K):
        oh = (iota0 == idx2[k:k + 1, :]).astype(_f32)
        xj = _dot(x1t, oh)
        h = u2 + _dot(w2v_ref[...], xj)
        acc2 = h if acc2 is None else jnp.maximum(acc2, h)
    x2t = acc2

    feat = jnp.concatenate([x1t, x2t], axis=0)
    out = _dot(wl_ref[...], feat) + bl_ref[...]
    pool_ref[0] = jnp.max(out, axis=1, keepdims=True)


def _head_kernel(p_ref, w1_ref, b1_ref, w2_ref, b2_ref, w3_ref, b3_ref,
                 out_ref):
    h = jnp.maximum(_dot(w1_ref[...], p_ref[...]) + b1_ref[...], 0.0)
    h = jnp.maximum(_dot(w2_ref[...], h) + b2_ref[...], 0.0)
    logit = _dot(w3_ref[...], h) + b3_ref[...]
    m = jnp.max(logit, axis=0, keepdims=True)
    s = logit - m
    out_ref[...] = s - jnp.log(jnp.sum(jnp.exp(s), axis=0, keepdims=True))


def _full(shape):
    nd = len(shape)
    return pl.BlockSpec(shape, lambda *_: (0,) * nd)


@functools.partial(jax.jit, static_argnames=())
def kernel(data, batch, W1a, b1a, W1b, b1b, W1c, b1c, W2, b2, Wl, bl,
           Wm1, bm1, Wm2, bm2, Wm3, bm3):
    del batch
    xt = data.reshape(B, P, 2).transpose(0, 2, 1)
    wu1 = (W1a[:2] - W1a[2:]).T
    wv1 = W1a[2:].T
    w2u = (W2[:64] - W2[64:]).T
    w2v = W2[64:].T

    col = lambda v: v[:, None]
    pooled = pl.pallas_call(
        _cloud_kernel,
        grid=(B,),
        in_specs=[
            pl.BlockSpec((1, 2, P), lambda b: (b, 0, 0)),
            _full((64, 2)), _full((64, 2)), _full((64, 1)),
            _full((64, 64)), _full((64, 1)),
            _full((64, 64)), _full((64, 1)),
            _full((128, 64)), _full((128, 64)), _full((128, 1)),
            _full((1024, 192)), _full((1024, 1)),
        ],
        out_specs=pl.BlockSpec((1, 1024, 1), lambda b: (b, 0, 0)),
        out_shape=jax.ShapeDtypeStruct((B, 1024, 1), _f32),
    )(xt, wu1, wv1, col(b1a), W1b.T, col(b1b), W1c.T, col(b1c),
      w2u, w2v, col(b2), Wl.T, col(bl))

    pooled_t = pooled[:, :, 0].T
    out_t = pl.pallas_call(
        _head_kernel,
        in_specs=[_full((1024, B)),
                  _full((512, 1024)), _full((512, 1)),
                  _full((256, 512)), _full((256, 1)),
                  _full((16, 256)), _full((16, 1))],
        out_specs=_full((16, B)),
        out_shape=jax.ShapeDtypeStruct((16, B), _f32),
    )(pooled_t, Wm1.T, col(bm1), Wm2.T, col(bm2), Wm3.T, col(bm3))
    return out_t.T
